# baseline (speedup 1.0000x reference)
import functools

import jax
import jax.numpy as jnp
from jax import lax
from jax.experimental import pallas as pl
from jax.experimental.pallas import tpu as pltpu
from jax.experimental.pallas import tpu_sc as plsc

NC = 2
NS = 16
L = 16
NW = NC * NS

CB = 32
IDX_W = 128


def _sigmoid(t):
    return 1.0 / (1.0 + jnp.exp(-t))


@functools.lru_cache(maxsize=None)
def _build(V, D, B, NNEG):
    assert B % (NW * CB) == 0 and D % L == 0
    bw = B // NW
    nchunk = bw // CB
    nneg_rows = CB * NNEG
    nj = nneg_rows // IDX_W
    assert nneg_rows % IDX_W == 0
    ngroups = CB // L

    mesh = plsc.VectorSubcoreMesh(core_axis_name="c", subcore_axis_name="s")

    @functools.partial(
        pl.kernel,
        out_type=(
            jax.ShapeDtypeStruct((B,), jnp.float32),
            jax.ShapeDtypeStruct((B * NNEG,), jnp.float32),
        ),
        mesh=mesh,
        compiler_params=pltpu.CompilerParams(
            needs_layout_passes=False, use_tc_tiling_on_sc=False),
        scratch_types=[
            pltpu.VMEM((CB,), jnp.int32),
            pltpu.VMEM((CB,), jnp.int32),
            pltpu.VMEM((nj, IDX_W), jnp.int32),
            pltpu.VMEM((CB, D), jnp.float32),
            pltpu.VMEM((CB, D), jnp.float32),
            pltpu.VMEM((nneg_rows, D), jnp.float32),
            pltpu.VMEM((CB,), jnp.float32),
            pltpu.VMEM((nneg_rows,), jnp.float32),
            pltpu.SemaphoreType.DMA,
        ],
    )
    def sc_kernel(ct_hbm, xt_hbm, cw_hbm, xw_hbm, nw_hbm,
                  pos_hbm, neg_hbm,
                  idxc, idxx, idxn, crows, xrows, nrows, posb, negb, sem):
        wid = lax.axis_index("s") * NC + lax.axis_index("c")
        lane = lax.iota(jnp.int32, L)

        def chunk_body(chunk, _):
            base = pl.multiple_of(wid * bw + chunk * CB, CB)
            nbase = pl.multiple_of(base * NNEG, CB * NNEG)
            pltpu.sync_copy(cw_hbm.at[pl.ds(base, CB)], idxc)
            pltpu.sync_copy(xw_hbm.at[pl.ds(base, CB)], idxx)
            for j in range(nj):
                pltpu.sync_copy(nw_hbm.at[pl.ds(nbase + j * IDX_W, IDX_W)],
                                idxn.at[j])
            cps = [
                pltpu.async_copy(ct_hbm.at[idxc], crows, sem),
                pltpu.async_copy(xt_hbm.at[idxx], xrows, sem),
            ]
            for j in range(nj):
                cps.append(pltpu.async_copy(
                    xt_hbm.at[idxn.at[j]],
                    nrows.at[pl.ds(j * IDX_W, IDX_W)], sem))
            for cp in cps:
                cp.wait()

            for g in range(ngroups):
                cidx = lane + g * L
                nrow0 = lane * NNEG + g * L * NNEG

                DU = 8

                def d_body(i, carry):
                    accp = carry[0]
                    accs = carry[1]
                    d0 = i * DU
                    for k in range(DU):
                        dsp = jnp.full((L,), d0 + k, jnp.int32)
                        c = plsc.load_gather(crows, [cidx, dsp])
                        x = plsc.load_gather(xrows, [cidx, dsp])
                        accp = accp + c * x
                        accs = tuple(
                            accs[n]
                            + plsc.load_gather(nrows, [nrow0 + n, dsp]) * c
                            for n in range(NNEG))
                    return (accp, accs)

                zero = jnp.zeros((L,), jnp.float32)
                accp, accs = lax.fori_loop(
                    0, D // DU, d_body, (zero, (zero,) * NNEG))

                posb[pl.ds(g * L, L)] = _sigmoid(accp)
                for n in range(NNEG):
                    plsc.store_scatter(negb, [nrow0 + n], _sigmoid(-accs[n]))

            pltpu.sync_copy(posb, pos_hbm.at[pl.ds(base, CB)])
            pltpu.sync_copy(negb, neg_hbm.at[pl.ds(nbase, nneg_rows)])
            return ()

        lax.fori_loop(0, nchunk, chunk_body, ())

    return sc_kernel


def kernel(center_table, context_table, center_words, context_words,
           negative_words):
    V, D = center_table.shape
    B = center_words.shape[0]
    NNEG = negative_words.shape[1]
    fn = _build(V, D, B, NNEG)
    pos, neg_flat = fn(
        center_table,
        context_table,
        center_words.astype(jnp.int32),
        context_words.astype(jnp.int32),
        negative_words.astype(jnp.int32).reshape(-1),
    )
    return pos, neg_flat.reshape(B, NNEG)


# speedup vs baseline: 1.0623x; 1.0623x over previous
import functools

import jax
import jax.numpy as jnp
from jax import lax
from jax.experimental import pallas as pl
from jax.experimental.pallas import tpu as pltpu
from jax.experimental.pallas import tpu_sc as plsc

NC = 2
NS = 16
L = 16
NW = NC * NS

CB = 32
IDX_W = 128
DU = 1


def _sigmoid(t):
    return 1.0 / (1.0 + jnp.exp(-t))


@functools.lru_cache(maxsize=None)
def _build(V, D, B, NNEG):
    assert B % (NW * CB) == 0 and D % L == 0
    bw = B // NW
    nchunk = bw // CB
    nneg_rows = CB * NNEG
    nj = nneg_rows // IDX_W
    assert nneg_rows % IDX_W == 0
    ngroups = CB // L

    mesh = plsc.VectorSubcoreMesh(core_axis_name="c", subcore_axis_name="s")

    @functools.partial(
        pl.kernel,
        out_type=(
            jax.ShapeDtypeStruct((B,), jnp.float32),
            jax.ShapeDtypeStruct((B * NNEG,), jnp.float32),
        ),
        mesh=mesh,
        compiler_params=pltpu.CompilerParams(
            needs_layout_passes=False, use_tc_tiling_on_sc=False),
        scratch_types=[
            pltpu.VMEM((2, CB), jnp.int32),
            pltpu.VMEM((2, CB), jnp.int32),
            pltpu.VMEM((2, nneg_rows), jnp.int32),
            pltpu.VMEM((2, CB, D), jnp.float32),
            pltpu.VMEM((2, CB, D), jnp.float32),
            pltpu.VMEM((2, nneg_rows, D), jnp.float32),
            pltpu.VMEM((2, CB), jnp.float32),
            pltpu.VMEM((2, nneg_rows), jnp.float32),
            pltpu.SemaphoreType.DMA,
            pltpu.SemaphoreType.DMA,
            pltpu.SemaphoreType.DMA,
            pltpu.SemaphoreType.DMA,
            pltpu.SemaphoreType.DMA,
            pltpu.SemaphoreType.DMA,
        ],
    )
    def sc_kernel(ct_hbm, xt_hbm, cw_hbm, xw_hbm, nw_hbm,
                  pos_hbm, neg_hbm,
                  idxc, idxx, idxn, crows, xrows, nrows, posb, negb,
                  semi0, semi1, semr0, semr1, semo0, semo1):
        semi = (semi0, semi1)
        semr = (semr0, semr1)
        semo = (semo0, semo1)
        wid = lax.axis_index("s") * NC + lax.axis_index("c")
        lane = lax.iota(jnp.int32, L)

        def chunk_base(i):
            return pl.multiple_of(wid * bw + i * CB, CB)

        def fire_idx(i):
            s = i % 2
            base = chunk_base(i)
            nbase = pl.multiple_of(base * NNEG, CB * NNEG)
            return [
                pltpu.async_copy(cw_hbm.at[pl.ds(base, CB)],
                                 idxc.at[s], semi[s]),
                pltpu.async_copy(xw_hbm.at[pl.ds(base, CB)],
                                 idxx.at[s], semi[s]),
                pltpu.async_copy(nw_hbm.at[pl.ds(nbase, nneg_rows)],
                                 idxn.at[s], semi[s]),
            ]

        def fire_rows(i):
            s = i % 2
            cps = [
                pltpu.async_copy(ct_hbm.at[idxc.at[s]], crows.at[s], semr[s]),
                pltpu.async_copy(xt_hbm.at[idxx.at[s]], xrows.at[s], semr[s]),
            ]
            for j in range(nj):
                cps.append(pltpu.async_copy(
                    xt_hbm.at[idxn.at[s, pl.ds(j * IDX_W, IDX_W)]],
                    nrows.at[s, pl.ds(j * IDX_W, IDX_W)], semr[s]))
            return cps

        def compute(i):
            s = i % 2
            cr, xr, nr = crows.at[s], xrows.at[s], nrows.at[s]

            def g_body(g, _):
                cidx = lane + g * L
                nrow0 = (lane + g * L) * NNEG

                def d_body(k, carry):
                    accp = carry[0]
                    accs = carry[1]
                    d0 = k * DU
                    for kk in range(DU):
                        dsp = jnp.full((L,), d0 + kk, jnp.int32)
                        c = plsc.load_gather(cr, [cidx, dsp])
                        x = plsc.load_gather(xr, [cidx, dsp])
                        accp = accp + c * x
                        accs = tuple(
                            accs[n]
                            + plsc.load_gather(nr, [nrow0 + n, dsp]) * c
                            for n in range(NNEG))
                    return (accp, accs)

                zero = jnp.zeros((L,), jnp.float32)
                accp, accs = lax.fori_loop(
                    0, D // DU, d_body, (zero, (zero,) * NNEG))
                plsc.store_scatter(posb.at[s], [cidx], _sigmoid(accp))
                for n in range(NNEG):
                    plsc.store_scatter(negb.at[s], [nrow0 + n],
                                       _sigmoid(-accs[n]))
                return ()

            lax.fori_loop(0, ngroups, g_body, ())

        def fire_out(i):
            s = i % 2
            base = chunk_base(i)
            nbase = pl.multiple_of(base * NNEG, CB * NNEG)
            return [
                pltpu.async_copy(posb.at[s], pos_hbm.at[pl.ds(base, CB)],
                                 semo[s]),
                pltpu.async_copy(negb.at[s],
                                 neg_hbm.at[pl.ds(nbase, nneg_rows)],
                                 semo[s]),
            ]

        def drain(cps):
            for cp in cps:
                cp.wait()

        d_idx, d_rows, d_out = {}, {}, {}
        d_idx[0] = fire_idx(0)
        d_idx[1] = fire_idx(1)
        drain(d_idx.pop(0))
        d_rows[0] = fire_rows(0)
        for i in range(nchunk):
            if i + 1 < nchunk:
                drain(d_idx.pop(i + 1))
                d_rows[i + 1] = fire_rows(i + 1)
            drain(d_rows.pop(i))
            if i + 2 < nchunk:
                d_idx[i + 2] = fire_idx(i + 2)
            if i - 2 in d_out:
                drain(d_out.pop(i - 2))
            compute(i)
            d_out[i] = fire_out(i)
        for k in sorted(d_out):
            drain(d_out[k])

    return sc_kernel


def kernel(center_table, context_table, center_words, context_words,
           negative_words):
    V, D = center_table.shape
    B = center_words.shape[0]
    NNEG = negative_words.shape[1]
    fn = _build(V, D, B, NNEG)
    pos, neg_flat = fn(
        center_table,
        context_table,
        center_words.astype(jnp.int32),
        context_words.astype(jnp.int32),
        negative_words.astype(jnp.int32).reshape(-1),
    )
    return pos, neg_flat.reshape(B, NNEG)
